# Initial kernel scaffold; baseline (speedup 1.0000x reference)
import jax
import jax.numpy as jnp
from jax.experimental import pallas as pl


def kernel(x, W1, b1, W2, b2):
    raise NotImplementedError("write your pallas kernel here")


# speedup vs baseline: 1.1804x; 1.1804x over previous
import functools

import jax
import jax.numpy as jnp
from jax.experimental import pallas as pl

B, D, N = 4, 128, 2048
BN = 256


def _attn_rows(q_ref, kv_ref, w_ref, b_ref):
    q = q_ref[0]
    kv = kv_ref[0]
    s = jax.lax.dot_general(
        q, kv, (((1,), (1,)), ((), ())),
        preferred_element_type=jnp.float32,
    ) * (1.0 / (D ** 0.5))
    m = jnp.max(s, axis=1, keepdims=True)
    e = jnp.exp(s - m)
    denom = jnp.sum(e, axis=1, keepdims=True)
    msg = jax.lax.dot_general(
        e, kv, (((1,), (0,)), ((), ())),
        preferred_element_type=jnp.float32,
    ) / denom
    out = jnp.dot(msg, w_ref[...], preferred_element_type=jnp.float32)
    return jnp.maximum(out + b_ref[...], 0.0)


def _layer1_body(q_ref, kv_ref, w_ref, b_ref, o_ref):
    o_ref[0] = _attn_rows(q_ref, kv_ref, w_ref, b_ref)


def _layer2_body(q_ref, kv_ref, w_ref, b_ref, o_ref):
    out = _attn_rows(q_ref, kv_ref, w_ref, b_ref)
    partial = (jnp.sum(out, axis=0, keepdims=True) * (1.0 / N))[None]
    nb = pl.program_id(1)

    @pl.when(nb == 0)
    def _():
        o_ref[...] = partial

    @pl.when(nb != 0)
    def _():
        o_ref[...] = o_ref[...] + partial


def _layer_specs():
    return [
        pl.BlockSpec((1, BN, D), lambda b, i: (b, i, 0)),
        pl.BlockSpec((1, N, D), lambda b, i: (b, 0, 0)),
        pl.BlockSpec((D, D), lambda b, i: (0, 0)),
        pl.BlockSpec((1, D), lambda b, i: (0, 0)),
    ]


@functools.partial(jax.jit, static_argnames=())
def kernel(x, W1, b1, W2, b2):
    h0 = jnp.transpose(x, (0, 2, 1))
    grid = (B, N // BN)

    h1 = pl.pallas_call(
        _layer1_body,
        grid=grid,
        in_specs=_layer_specs(),
        out_specs=pl.BlockSpec((1, BN, D), lambda b, i: (b, i, 0)),
        out_shape=jax.ShapeDtypeStruct((B, N, D), jnp.float32),
    )(h0, h0, W1, b1.reshape(1, D))

    pooled = pl.pallas_call(
        _layer2_body,
        grid=grid,
        in_specs=_layer_specs(),
        out_specs=pl.BlockSpec((1, 1, D), lambda b, i: (b, 0, 0)),
        out_shape=jax.ShapeDtypeStruct((B, 1, D), jnp.float32),
    )(h1, h1, W2, b2.reshape(1, D))

    return pooled[:, 0, :]


# speedup vs baseline: 1.2687x; 1.0748x over previous
import functools

import jax
import jax.numpy as jnp
from jax.experimental import pallas as pl

B, D, N = 4, 128, 2048
BN = 256


def _attn_rows(q_ref, kv_ref, w_ref, b_ref):
    q = q_ref[0].astype(jnp.bfloat16)
    kv = kv_ref[0].astype(jnp.bfloat16)
    s = jax.lax.dot_general(
        q, kv, (((1,), (1,)), ((), ())),
        preferred_element_type=jnp.float32,
    ) * (1.0 / (D ** 0.5))
    m = jnp.max(s, axis=1, keepdims=True)
    e = jnp.exp(s - m)
    denom = jnp.sum(e, axis=1, keepdims=True)
    msg = jax.lax.dot_general(
        e.astype(jnp.bfloat16), kv, (((1,), (0,)), ((), ())),
        preferred_element_type=jnp.float32,
    ) / denom
    out = jnp.dot(msg, w_ref[...], preferred_element_type=jnp.float32)
    return jnp.maximum(out + b_ref[...], 0.0)


def _layer1_body(q_ref, kv_ref, w_ref, b_ref, o_ref):
    o_ref[0] = _attn_rows(q_ref, kv_ref, w_ref, b_ref)


def _layer2_body(q_ref, kv_ref, w_ref, b_ref, o_ref):
    out = _attn_rows(q_ref, kv_ref, w_ref, b_ref)
    partial = (jnp.sum(out, axis=0, keepdims=True) * (1.0 / N))[None]
    nb = pl.program_id(1)

    @pl.when(nb == 0)
    def _():
        o_ref[...] = partial

    @pl.when(nb != 0)
    def _():
        o_ref[...] = o_ref[...] + partial


def _layer_specs():
    return [
        pl.BlockSpec((1, BN, D), lambda b, i: (b, i, 0)),
        pl.BlockSpec((1, N, D), lambda b, i: (b, 0, 0)),
        pl.BlockSpec((D, D), lambda b, i: (0, 0)),
        pl.BlockSpec((1, D), lambda b, i: (0, 0)),
    ]


@functools.partial(jax.jit, static_argnames=())
def kernel(x, W1, b1, W2, b2):
    h0 = jnp.transpose(x, (0, 2, 1))
    grid = (B, N // BN)

    h1 = pl.pallas_call(
        _layer1_body,
        grid=grid,
        in_specs=_layer_specs(),
        out_specs=pl.BlockSpec((1, BN, D), lambda b, i: (b, i, 0)),
        out_shape=jax.ShapeDtypeStruct((B, N, D), jnp.float32),
    )(h0, h0, W1, b1.reshape(1, D))

    pooled = pl.pallas_call(
        _layer2_body,
        grid=grid,
        in_specs=_layer_specs(),
        out_specs=pl.BlockSpec((1, 1, D), lambda b, i: (b, 0, 0)),
        out_shape=jax.ShapeDtypeStruct((B, 1, D), jnp.float32),
    )(h1, h1, W2, b2.reshape(1, D))

    return pooled[:, 0, :]
